# baseline (speedup 1.0000x reference)
import jax
import jax.numpy as jnp
from jax import lax
from jax.experimental import pallas as pl
from jax.experimental.pallas import tpu as pltpu

N = 10000
D = 512
HID = 1024
K = 64
K_SORT = 69
CHUNK = 2048
NCHUNK = 5

NEG_INF = float("-inf")


def _argmax_stable(v, idx):
    def comb(v1, i1, v2, i2):
        keep = (v1 > v2) | ((v1 == v2) & (i1 <= i2))
        return jnp.where(keep, v1, v2), jnp.where(keep, i1, i2)

    w = v.shape[1]
    while w > 1:
        h = w // 2
        v, idx = comb(v[:, :h], idx[:, :h], v[:, h:], idx[:, h:])
        w = h
    s = v.shape[0]
    while s > 1:
        h = s // 2
        v, idx = comb(v[:h], idx[:h], v[h:], idx[h:])
        s = h
    return v, idx


def _fused_body(hs_ref, w_ref, b_ref, nodes_ref, nodes_any,
                idx_ref, score_ref, chose_ref, h_s, lg_s, rows_s, sem):
    i = pl.program_id(0)

    @pl.when(i == 0)
    def _():
        hs8 = jnp.broadcast_to(hs_ref[...], (8, HID))
        hcol = lax.dot_general(
            w_ref[...].astype(jnp.bfloat16), hs8.astype(jnp.bfloat16),
            (((1,), (1,)), ((), ())),
            preferred_element_type=jnp.float32,
        )
        h_s[...] = hcol + jnp.transpose(b_ref[...])

    lg8 = lax.dot_general(
        nodes_ref[...].astype(jnp.bfloat16), h_s[...].astype(jnp.bfloat16),
        (((1,), (0,)), ((), ())),
        preferred_element_type=jnp.float32,
    )
    lgT = jnp.transpose(lg8)
    sub = lax.broadcasted_iota(jnp.int32, (8, CHUNK), 0)
    lg_s[...] = jnp.where(sub == i, lgT, lg_s[...])

    @pl.when(i == NCHUNK - 1)
    def _():
        lane = lax.broadcasted_iota(jnp.int32, (8, CHUNK), 1)
        flat = sub * CHUNK + lane
        x = jnp.where(flat < N, lg_s[...], NEG_INF)
        m = jnp.max(x)
        denom = jnp.sum(jnp.exp(x - m))
        lane128 = lax.broadcasted_iota(jnp.int32, (1, 128), 1)

        idxs = jnp.zeros((1, 128), jnp.int32)
        vals = jnp.full((1, 128), NEG_INF)
        dmas = []
        for j in range(K_SORT):
            cm, ij = _argmax_stable(x, flat)
            idxs = jnp.where(lane128 == j, ij, idxs)
            vals = jnp.where(lane128 == j, cm, vals)
            x = jnp.where(flat == ij, NEG_INF, x)
            if j < K:
                cp = pltpu.make_async_copy(
                    nodes_any.at[pl.ds(ij[0, 0], 1)],
                    rows_s.at[pl.ds(j, 1)],
                    sem,
                )
                cp.start()
                dmas.append(cp)

        idx_ref[...] = idxs
        sc = jnp.exp(vals - m) / denom
        score_ref[...] = sc
        for cp in dmas:
            cp.wait()
        sc64 = jnp.transpose(sc)[:K]
        chose_ref[...] = rows_s[...] * sc64


def kernel(nodes, hidden_state, W, b):
    b2 = b.reshape(1, D)
    idx128, score128, chose = pl.pallas_call(
        _fused_body,
        grid=(NCHUNK,),
        in_specs=[
            pl.BlockSpec((1, HID), lambda i: (0, 0)),
            pl.BlockSpec((D, HID), lambda i: (0, 0)),
            pl.BlockSpec((1, D), lambda i: (0, 0)),
            pl.BlockSpec((CHUNK, D), lambda i: (i, 0)),
            pl.BlockSpec(memory_space=pl.ANY),
        ],
        out_specs=(
            pl.BlockSpec((1, 128), lambda i: (0, 0)),
            pl.BlockSpec((1, 128), lambda i: (0, 0)),
            pl.BlockSpec((K, D), lambda i: (0, 0)),
        ),
        out_shape=(
            jax.ShapeDtypeStruct((1, 128), jnp.int32),
            jax.ShapeDtypeStruct((1, 128), jnp.float32),
            jax.ShapeDtypeStruct((K, D), jnp.float32),
        ),
        scratch_shapes=[
            pltpu.VMEM((D, 8), jnp.float32),
            pltpu.VMEM((8, CHUNK), jnp.float32),
            pltpu.VMEM((K, D), jnp.float32),
            pltpu.SemaphoreType.DMA,
        ],
    )(hidden_state, W, b2, nodes, nodes)
    sort_nodes_index = idx128[0, :K_SORT]
    topk_scores = score128[0, :K]
    return chose.reshape(1, K * D), topk_scores, sort_nodes_index


# speedup vs baseline: 1.0181x; 1.0181x over previous
import jax
import jax.numpy as jnp
from jax import lax
from jax.experimental import pallas as pl
from jax.experimental.pallas import tpu as pltpu

N = 10000
D = 512
HID = 1024
K = 64
K_SORT = 69
CHUNK = 2048
NCHUNK = 5

NEG_INF = float("-inf")


def _argmax_stable(v, idx):
    def comb(v1, i1, v2, i2):
        keep = (v1 > v2) | ((v1 == v2) & (i1 <= i2))
        return jnp.where(keep, v1, v2), jnp.where(keep, i1, i2)

    w = v.shape[1]
    while w > 128:
        h = w // 2
        v, idx = comb(v[:, :h], idx[:, :h], v[:, h:], idx[:, h:])
        w = h
    for off in (4, 2, 1):
        v, idx = comb(v, idx, pltpu.roll(v, off, 0), pltpu.roll(idx, off, 0))
    for off in (64, 32, 16, 8, 4, 2, 1):
        v, idx = comb(v, idx, pltpu.roll(v, off, 1), pltpu.roll(idx, off, 1))
    return v, idx


def _fused_body(hs_ref, w_ref, b_ref, nodes_ref, nodes_any,
                idx_ref, score_ref, chose_ref, h_s, lg_s, rows_s, sem):
    i = pl.program_id(0)

    @pl.when(i == 0)
    def _():
        hs8 = jnp.broadcast_to(hs_ref[...], (8, HID))
        hcol = lax.dot_general(
            w_ref[...].astype(jnp.bfloat16), hs8.astype(jnp.bfloat16),
            (((1,), (1,)), ((), ())),
            preferred_element_type=jnp.float32,
        )
        h_s[...] = hcol + jnp.transpose(b_ref[...])

    lg8 = lax.dot_general(
        nodes_ref[...].astype(jnp.bfloat16), h_s[...].astype(jnp.bfloat16),
        (((1,), (0,)), ((), ())),
        preferred_element_type=jnp.float32,
    )
    lgT = jnp.transpose(lg8)
    sub = lax.broadcasted_iota(jnp.int32, (8, CHUNK), 0)
    lg_s[...] = jnp.where(sub == i, lgT, lg_s[...])

    @pl.when(i == NCHUNK - 1)
    def _():
        lane = lax.broadcasted_iota(jnp.int32, (8, CHUNK), 1)
        flat = sub * CHUNK + lane
        x = jnp.where(flat < N, lg_s[...], NEG_INF)
        m = jnp.max(x)
        denom = jnp.sum(jnp.exp(x - m))
        lane128 = lax.broadcasted_iota(jnp.int32, (1, 128), 1)

        idxs = jnp.zeros((1, 128), jnp.int32)
        vals = jnp.full((1, 128), NEG_INF)
        dmas = []
        for j in range(K_SORT):
            cm, ij = _argmax_stable(x, flat)
            idxs = jnp.where(lane128 == j, ij[0:1], idxs)
            vals = jnp.where(lane128 == j, cm[0:1], vals)
            ij_w = jnp.concatenate([ij] * (CHUNK // 128), axis=1)
            x = jnp.where(flat == ij_w, NEG_INF, x)
            if j < K:
                cp = pltpu.make_async_copy(
                    nodes_any.at[pl.ds(ij[0, 0], 1)],
                    rows_s.at[pl.ds(j, 1)],
                    sem,
                )
                cp.start()
                dmas.append(cp)

        idx_ref[...] = idxs
        sc = jnp.exp(vals - m) / denom
        score_ref[...] = sc
        for cp in dmas:
            cp.wait()
        sc64 = jnp.transpose(sc)[:K]
        chose_ref[...] = rows_s[...] * sc64


def kernel(nodes, hidden_state, W, b):
    b2 = b.reshape(1, D)
    idx128, score128, chose = pl.pallas_call(
        _fused_body,
        grid=(NCHUNK,),
        in_specs=[
            pl.BlockSpec((1, HID), lambda i: (0, 0)),
            pl.BlockSpec((D, HID), lambda i: (0, 0)),
            pl.BlockSpec((1, D), lambda i: (0, 0)),
            pl.BlockSpec((CHUNK, D), lambda i: (i, 0)),
            pl.BlockSpec(memory_space=pl.ANY),
        ],
        out_specs=(
            pl.BlockSpec((1, 128), lambda i: (0, 0)),
            pl.BlockSpec((1, 128), lambda i: (0, 0)),
            pl.BlockSpec((K, D), lambda i: (0, 0)),
        ),
        out_shape=(
            jax.ShapeDtypeStruct((1, 128), jnp.int32),
            jax.ShapeDtypeStruct((1, 128), jnp.float32),
            jax.ShapeDtypeStruct((K, D), jnp.float32),
        ),
        scratch_shapes=[
            pltpu.VMEM((D, 8), jnp.float32),
            pltpu.VMEM((8, CHUNK), jnp.float32),
            pltpu.VMEM((K, D), jnp.float32),
            pltpu.SemaphoreType.DMA,
        ],
    )(hidden_state, W, b2, nodes, nodes)
    sort_nodes_index = idx128[0, :K_SORT]
    topk_scores = score128[0, :K]
    return chose.reshape(1, K * D), topk_scores, sort_nodes_index


# speedup vs baseline: 1.4544x; 1.4285x over previous
import jax
import jax.numpy as jnp
from jax import lax
from jax.experimental import pallas as pl
from jax.experimental.pallas import tpu as pltpu

N = 10000
D = 512
HID = 1024
K = 64
K_SORT = 69
CHUNK = 2048
NCHUNK = 5

NEG_INF = float("-inf")


def _pbefore(v1, i1, v2, i2):
    return (v1 > v2) | ((v1 == v2) & (i1 <= i2))


def _top2_stable(x, flat):
    def better(v1, i1, v2, i2):
        k = _pbefore(v1, i1, v2, i2)
        return jnp.where(k, v1, v2), jnp.where(k, i1, i2)

    def merge(a, b):
        a1v, a1i, a2v, a2i = a
        b1v, b1i, b2v, b2i = b
        t = _pbefore(a1v, a1i, b1v, b1i)
        fv = jnp.where(t, a1v, b1v)
        fi = jnp.where(t, a1i, b1i)
        c1v, c1i = better(a2v, a2i, b1v, b1i)
        c2v, c2i = better(a1v, a1i, b2v, b2i)
        sv = jnp.where(t, c1v, c2v)
        si = jnp.where(t, c1i, c2i)
        return (fv, fi, sv, si)

    s = (x, flat,
         jnp.full(x.shape, NEG_INF),
         jnp.full(x.shape, jnp.int32(2**31 - 1)))
    w = x.shape[1]
    while w > 128:
        h = w // 2
        s = merge(tuple(a[:, :h] for a in s), tuple(a[:, h:] for a in s))
        w = h
    for off in (4, 2, 1):
        s = merge(s, tuple(pltpu.roll(a, off, 0) for a in s))
    for off in (64, 32, 16, 8, 4, 2, 1):
        s = merge(s, tuple(pltpu.roll(a, off, 1) for a in s))
    return s


def _fused_body(hs_ref, w_ref, b_ref, nodes_ref, nodes_any,
                idx_ref, score_ref, chose_ref, h_s, lg_s, rows_s, sem):
    i = pl.program_id(0)

    @pl.when(i == 0)
    def _():
        hs8 = jnp.broadcast_to(hs_ref[...], (8, HID))
        hcol = lax.dot_general(
            w_ref[...].astype(jnp.bfloat16), hs8.astype(jnp.bfloat16),
            (((1,), (1,)), ((), ())),
            preferred_element_type=jnp.float32,
        )
        h_s[...] = hcol + jnp.transpose(b_ref[...])

    lg8 = lax.dot_general(
        nodes_ref[...].astype(jnp.bfloat16), h_s[...].astype(jnp.bfloat16),
        (((1,), (0,)), ((), ())),
        preferred_element_type=jnp.float32,
    )
    lgT = jnp.transpose(lg8)
    sub = lax.broadcasted_iota(jnp.int32, (8, CHUNK), 0)
    lg_s[...] = jnp.where(sub == i, lgT, lg_s[...])

    @pl.when(i == NCHUNK - 1)
    def _():
        lane = lax.broadcasted_iota(jnp.int32, (8, CHUNK), 1)
        flat = sub * CHUNK + lane
        x = jnp.where(flat < N, lg_s[...], NEG_INF)
        m = jnp.max(x)
        denom = jnp.sum(jnp.exp(x - m))
        lane128 = lax.broadcasted_iota(jnp.int32, (1, 128), 1)

        idxs = jnp.zeros((1, 128), jnp.int32)
        vals = jnp.full((1, 128), NEG_INF)
        dmas = []

        def start_row_dma(j, ij):
            cp = pltpu.make_async_copy(
                nodes_any.at[pl.ds(ij[0, 0], 1)],
                rows_s.at[pl.ds(j, 1)],
                sem,
            )
            cp.start()
            dmas.append(cp)

        for r in range((K_SORT + 1) // 2):
            j1, j2 = 2 * r, 2 * r + 1
            fv, fi, sv, si = _top2_stable(x, flat)
            idxs = jnp.where(lane128 == j1, fi[0:1], idxs)
            vals = jnp.where(lane128 == j1, fv[0:1], vals)
            fi_w = jnp.concatenate([fi] * (CHUNK // 128), axis=1)
            if j2 < K_SORT:
                idxs = jnp.where(lane128 == j2, si[0:1], idxs)
                vals = jnp.where(lane128 == j2, sv[0:1], vals)
                si_w = jnp.concatenate([si] * (CHUNK // 128), axis=1)
                x = jnp.where((flat == fi_w) | (flat == si_w), NEG_INF, x)
            else:
                x = jnp.where(flat == fi_w, NEG_INF, x)
            if j1 < K:
                start_row_dma(j1, fi)
            if j2 < K:
                start_row_dma(j2, si)

        idx_ref[...] = idxs
        sc = jnp.exp(vals - m) / denom
        score_ref[...] = sc
        for cp in dmas:
            cp.wait()
        sc64 = jnp.transpose(sc)[:K]
        chose_ref[...] = rows_s[...] * sc64


def kernel(nodes, hidden_state, W, b):
    b2 = b.reshape(1, D)
    idx128, score128, chose = pl.pallas_call(
        _fused_body,
        grid=(NCHUNK,),
        in_specs=[
            pl.BlockSpec((1, HID), lambda i: (0, 0)),
            pl.BlockSpec((D, HID), lambda i: (0, 0)),
            pl.BlockSpec((1, D), lambda i: (0, 0)),
            pl.BlockSpec((CHUNK, D), lambda i: (i, 0)),
            pl.BlockSpec(memory_space=pl.ANY),
        ],
        out_specs=(
            pl.BlockSpec((1, 128), lambda i: (0, 0)),
            pl.BlockSpec((1, 128), lambda i: (0, 0)),
            pl.BlockSpec((K, D), lambda i: (0, 0)),
        ),
        out_shape=(
            jax.ShapeDtypeStruct((1, 128), jnp.int32),
            jax.ShapeDtypeStruct((1, 128), jnp.float32),
            jax.ShapeDtypeStruct((K, D), jnp.float32),
        ),
        scratch_shapes=[
            pltpu.VMEM((D, 8), jnp.float32),
            pltpu.VMEM((8, CHUNK), jnp.float32),
            pltpu.VMEM((K, D), jnp.float32),
            pltpu.SemaphoreType.DMA,
        ],
    )(hidden_state, W, b2, nodes, nodes)
    sort_nodes_index = idx128[0, :K_SORT]
    topk_scores = score128[0, :K]
    return chose.reshape(1, K * D), topk_scores, sort_nodes_index
